# baseline (speedup 1.0000x reference)
import jax
import jax.numpy as jnp
from jax import lax
from jax.experimental import pallas as pl
from jax.experimental.pallas import tpu as pltpu, tpu_sc as plsc

B = 16
C = 3
H = 512
W = 512
HW = H * W
NWORK = 32
PIX_PER_W = B * HW // NWORK
CHUNK = 1024
NCHUNK = PIX_PER_W // CHUNK
GRP = CHUNK // 16
JBLK = CHUNK // 128


def _sc_warp(coef_hbm, depth_hbm, imgn_hbm, out_hbm, imgf_hbm,
             coef_v, dep_a, dep_b, ia_v, ib_v, wa_v, wb_v,
             vva_v, vvb_v, o0_v, o1_v, o2_v, pk_v, sem):
    cid = lax.axis_index("c")
    sid = lax.axis_index("s")
    wid = cid * 16 + sid
    b = wid // 2
    pix0 = (wid % 2) * PIX_PER_W

    pltpu.sync_copy(coef_hbm.at[pl.ds(b * 32, 32)], coef_v)

    def _splat(k):
        return plsc.load_gather(coef_v, [jnp.full((16,), k, jnp.int32)])

    ki00 = _splat(0)
    ki02 = _splat(1)
    ki11 = _splat(2)
    ki12 = _splat(3)
    t = [_splat(4 + i) for i in range(12)]
    k00 = _splat(16)
    k02 = _splat(17)
    k11 = _splat(18)
    k12 = _splat(19)

    lane = lax.iota(jnp.int32, 16)
    vals_a = tuple(vva_v)
    vals_b = tuple(vvb_v)
    outs = (o0_v, o1_v, o2_v)

    def repack(ci, _):
        pbase = pix0 + ci * CHUNK
        for c in range(3):
            pltpu.sync_copy(
                imgn_hbm.at[pl.ds((b * C + c) * HW + pbase, CHUNK)], outs[c])

        def igrp(g, _g):
            row = jnp.full((16,), g * 16, jnp.int32) + lane
            for c in range(3):
                v = outs[c][pl.ds(g * 16, 16)]
                plsc.store_scatter(
                    pk_v, [row, jnp.full((16,), c, jnp.int32)], v)
            return _g

        lax.fori_loop(0, GRP, igrp, None)
        pltpu.sync_copy(pk_v, imgf_hbm.at[pl.ds(b * HW + pbase, CHUNK)])
        return _

    lax.fori_loop(0, NCHUNK, repack, None)
    plsc.subcore_barrier()

    def pass1(ci, dep_v, idx_t, w_v):
        pbase = pix0 + ci * CHUNK
        pltpu.sync_copy(depth_hbm.at[pl.ds(b * HW + pbase, CHUNK)], dep_v)

        def grp(g, _):
            pid = jnp.full((16,), pbase + g * 16, jnp.int32) + lane
            vrow = lax.shift_right_logical(pid, 9).astype(jnp.float32)
            ucol = (pid & (W - 1)).astype(jnp.float32)
            d = dep_v[pl.ds(g * 16, 16)]
            x = ki00 * ucol + ki02
            y = ki11 * vrow + ki12
            dx = d * x
            dy = d * y
            r0 = t[0] * dx + t[1] * dy + t[2] * d + t[3]
            r1 = t[4] * dx + t[5] * dy + t[6] * d + t[7]
            r2 = t[8] * dx + t[9] * dy + t[10] * d + t[11]
            den = r2 + 1e-16
            u = (k00 * r0 + k02 * r2) / den
            v = (k11 * r1 + k12 * r2) / den
            us = jnp.minimum(jnp.maximum(u, -1e9), 1e9)
            vs = jnp.minimum(jnp.maximum(v, -1e9), 1e9)
            ut = us.astype(jnp.int32).astype(jnp.float32)
            vt = vs.astype(jnp.int32).astype(jnp.float32)
            uf = ut - jnp.where(ut > us, 1.0, 0.0).astype(jnp.float32)
            vf = vt - jnp.where(vt > vs, 1.0, 0.0).astype(jnp.float32)
            u0f = jnp.minimum(jnp.maximum(uf, 0.0), float(W - 1))
            u1f = jnp.minimum(jnp.maximum(uf + 1.0, 0.0), float(W - 1))
            v0f = jnp.minimum(jnp.maximum(vf, 0.0), float(H - 1))
            v1f = jnp.minimum(jnp.maximum(vf + 1.0, 0.0), float(H - 1))
            u0 = u0f.astype(jnp.int32)
            u1 = u1f.astype(jnp.int32)
            gb = jnp.full((16,), b * HW, jnp.int32)
            base0 = gb + v0f.astype(jnp.int32) * W
            base1 = gb + v1f.astype(jnp.int32) * W
            idx_t[0][pl.ds(g * 16, 16)] = base0 + u0
            idx_t[1][pl.ds(g * 16, 16)] = base1 + u0
            idx_t[2][pl.ds(g * 16, 16)] = base0 + u1
            idx_t[3][pl.ds(g * 16, 16)] = base1 + u1
            w_v[pl.ds(g * 16, 16)] = (v1f - v) * (u1f - u)
            w_v[pl.ds(CHUNK + g * 16, 16)] = (v - v0f) * (u1f - u)
            w_v[pl.ds(2 * CHUNK + g * 16, 16)] = (v1f - v) * (u - u0f)
            w_v[pl.ds(3 * CHUNK + g * 16, 16)] = (v - v0f) * (u - u0f)
            return _

        lax.fori_loop(0, GRP, grp, None)

    def fire(idx_t, val_t):
        def fj(j, _):
            for ti in range(4):
                isl = idx_t[ti].at[pl.ds(j * 128, 128)]
                pltpu.async_copy(
                    imgf_hbm.at[isl],
                    val_t[ti].at[pl.ds(j * 128, 128)], sem)
            return _

        lax.fori_loop(0, JBLK, fj, None)

    def drain(idx_t, val_t):
        def dj(j, _):
            for ti in range(4):
                isl = idx_t[ti].at[pl.ds(j * 128, 128)]
                pltpu.make_async_copy(
                    imgf_hbm.at[isl],
                    val_t[ti].at[pl.ds(j * 128, 128)], sem).wait()
            return _

        lax.fori_loop(0, JBLK, dj, None)

    def pass2(ci, w_v, val_t):
        pbase = pix0 + ci * CHUNK

        def grp(g, _):
            s = pl.ds(g * 16, 16)
            wta = w_v[s]
            wtb = w_v[pl.ds(CHUNK + g * 16, 16)]
            wtc = w_v[pl.ds(2 * CHUNK + g * 16, 16)]
            wtd = w_v[pl.ds(3 * CHUNK + g * 16, 16)]
            rloc = jnp.full((16,), g * 16, jnp.int32) + lane
            for c in range(3):
                csp = jnp.full((16,), c, jnp.int32)
                pa = plsc.load_gather(val_t[0], [rloc, csp])
                pb = plsc.load_gather(val_t[1], [rloc, csp])
                pc = plsc.load_gather(val_t[2], [rloc, csp])
                pd = plsc.load_gather(val_t[3], [rloc, csp])
                outs[c][s] = wta * pa + wtb * pb + wtc * pc + wtd * pd
            return _

        lax.fori_loop(0, GRP, grp, None)
        for c in range(3):
            pltpu.sync_copy(
                outs[c], out_hbm.at[pl.ds((b * C + c) * HW + pbase, CHUNK)])

    idxs_a = tuple(ia_v)
    idxs_b = tuple(ib_v)

    pass1(0, dep_a, idxs_a, wa_v)
    fire(idxs_a, vals_a)

    def two_chunks(i, _):
        k = 2 * i

        @pl.when(k + 1 < NCHUNK)
        def _a():
            pass1(k + 1, dep_b, idxs_b, wb_v)

        drain(idxs_a, vals_a)

        @pl.when(k + 1 < NCHUNK)
        def _b():
            fire(idxs_b, vals_b)

        pass2(k, wa_v, vals_a)

        @pl.when(k + 2 < NCHUNK)
        def _c():
            pass1(k + 2, dep_a, idxs_a, wa_v)
            fire(idxs_a, vals_a)

        @pl.when(k + 1 < NCHUNK)
        def _d():
            drain(idxs_b, vals_b)
            pass2(k + 1, wb_v, vals_b)

        return _

    lax.fori_loop(0, (NCHUNK + 1) // 2, two_chunks, None)


def kernel(img, depth, T, K, K_inv):
    imgn = img.reshape(B * C * HW)
    depth2 = depth.reshape(B * HW)
    kiv = jnp.stack([K_inv[0, 0], K_inv[0, 2], K_inv[1, 1], K_inv[1, 2]])
    kv = jnp.stack([K[0, 0], K[0, 2], K[1, 1], K[1, 2]])
    coefs = jnp.concatenate(
        [jnp.broadcast_to(kiv[None, :], (B, 4)),
         T.reshape(B, 12),
         jnp.broadcast_to(kv[None, :], (B, 4)),
         jnp.zeros((B, 12), jnp.float32)], axis=1).reshape(B * 32)

    mesh = plsc.VectorSubcoreMesh(core_axis_name="c", subcore_axis_name="s")
    fn = pl.kernel(
        _sc_warp, mesh=mesh,
        out_type=(jax.ShapeDtypeStruct((B * C * HW,), jnp.float32),
                  jax.ShapeDtypeStruct((B * HW, C), jnp.float32)),
        scratch_types=(
            [pltpu.VMEM((32,), jnp.float32)]
            + [pltpu.VMEM((CHUNK,), jnp.float32)] * 2
            + [[pltpu.VMEM((CHUNK,), jnp.int32)] * 4] * 2
            + [pltpu.VMEM((4 * CHUNK,), jnp.float32)] * 2
            + [[pltpu.VMEM((CHUNK, C), jnp.float32)] * 4] * 2
            + [pltpu.VMEM((CHUNK,), jnp.float32)] * 3
            + [pltpu.VMEM((CHUNK, C), jnp.float32)]
            + [pltpu.SemaphoreType.DMA]
        ),
        compiler_params=pltpu.CompilerParams(
            needs_layout_passes=False, use_tc_tiling_on_sc=False),
    )
    out, _ = fn(coefs, depth2, imgn)
    return out.reshape(B, C, H, W)


# speedup vs baseline: 1.2574x; 1.2574x over previous
import jax
import jax.numpy as jnp
from jax import lax
from jax.experimental import pallas as pl
from jax.experimental.pallas import tpu as pltpu, tpu_sc as plsc

B = 16
C = 3
H = 512
W = 512
HW = H * W
NWORK = 32
PIX_PER_W = B * HW // NWORK
CHUNK = 2048
NCHUNK = PIX_PER_W // CHUNK
GRP = CHUNK // 16
JBLK = CHUNK // 128


def _sc_warp(u_hbm, v_hbm, imgn_hbm, out_hbm, imgf_hbm,
             ua_v, uc_a, uc_b, vc_a, vc_b, ia_v, ib_v, wa_v, wb_v,
             va_v, vb_v, vc_v, vd_v, o0_v, o1_v, o2_v, pk_v, sem):
    cid = lax.axis_index("c")
    sid = lax.axis_index("s")
    wid = cid * 16 + sid
    b = wid // 2
    pix0 = (wid % 2) * PIX_PER_W

    lane = lax.iota(jnp.int32, 16)
    vals = (va_v, vb_v, vc_v, vd_v)
    outs = (o0_v, o1_v, o2_v)

    def repack(ci, _):
        pbase = pix0 + ci * CHUNK
        for c in range(3):
            pltpu.sync_copy(
                imgn_hbm.at[pl.ds((b * C + c) * HW + pbase, CHUNK)], outs[c])

        def igrp(g, _g):
            row = jnp.full((16,), g * 16, jnp.int32) + lane
            for c in range(3):
                v = outs[c][pl.ds(g * 16, 16)]
                plsc.store_scatter(
                    pk_v, [row, jnp.full((16,), c, jnp.int32)], v)
            return _g

        lax.fori_loop(0, GRP, igrp, None)
        pltpu.sync_copy(pk_v, imgf_hbm.at[pl.ds(b * HW + pbase, CHUNK)])
        return _

    lax.fori_loop(0, NCHUNK, repack, None)
    plsc.subcore_barrier()

    def pass1(ci, ubufs, idx_t, w_v):
        pbase = pix0 + ci * CHUNK
        u_v, v_v = ubufs
        pltpu.sync_copy(u_hbm.at[pl.ds(b * HW + pbase, CHUNK)], u_v)
        pltpu.sync_copy(v_hbm.at[pl.ds(b * HW + pbase, CHUNK)], v_v)

        def grp(g, _):
            u = u_v[pl.ds(g * 16, 16)]
            v = v_v[pl.ds(g * 16, 16)]
            us = jnp.minimum(jnp.maximum(u, -1e9), 1e9)
            vs = jnp.minimum(jnp.maximum(v, -1e9), 1e9)
            ut = us.astype(jnp.int32).astype(jnp.float32)
            vt = vs.astype(jnp.int32).astype(jnp.float32)
            uf = ut - jnp.where(ut > us, 1.0, 0.0).astype(jnp.float32)
            vf = vt - jnp.where(vt > vs, 1.0, 0.0).astype(jnp.float32)
            u0f = jnp.minimum(jnp.maximum(uf, 0.0), float(W - 1))
            u1f = jnp.minimum(jnp.maximum(uf + 1.0, 0.0), float(W - 1))
            v0f = jnp.minimum(jnp.maximum(vf, 0.0), float(H - 1))
            v1f = jnp.minimum(jnp.maximum(vf + 1.0, 0.0), float(H - 1))
            u0 = u0f.astype(jnp.int32)
            u1 = u1f.astype(jnp.int32)
            gb = jnp.full((16,), b * HW, jnp.int32)
            base0 = gb + v0f.astype(jnp.int32) * W
            base1 = gb + v1f.astype(jnp.int32) * W
            idx_t[0][pl.ds(g * 16, 16)] = base0 + u0
            idx_t[1][pl.ds(g * 16, 16)] = base1 + u0
            idx_t[2][pl.ds(g * 16, 16)] = base0 + u1
            idx_t[3][pl.ds(g * 16, 16)] = base1 + u1
            w_v[pl.ds(g * 16, 16)] = (v1f - v) * (u1f - u)
            w_v[pl.ds(CHUNK + g * 16, 16)] = (v - v0f) * (u1f - u)
            w_v[pl.ds(2 * CHUNK + g * 16, 16)] = (v1f - v) * (u - u0f)
            w_v[pl.ds(3 * CHUNK + g * 16, 16)] = (v - v0f) * (u - u0f)
            return _

        lax.fori_loop(0, GRP, grp, None)

    def fire(idx_t):
        def fj(j, _):
            for ti in range(4):
                isl = idx_t[ti].at[pl.ds(j * 128, 128)]
                pltpu.async_copy(
                    imgf_hbm.at[isl],
                    vals[ti].at[pl.ds(j * 128, 128)], sem)
            return _

        lax.fori_loop(0, JBLK, fj, None)

    def drain(idx_t):
        def dj(j, _):
            for ti in range(4):
                isl = idx_t[ti].at[pl.ds(j * 128, 128)]
                pltpu.make_async_copy(
                    imgf_hbm.at[isl],
                    vals[ti].at[pl.ds(j * 128, 128)], sem).wait()
            return _

        lax.fori_loop(0, JBLK, dj, None)

    def pass2(ci, w_v):
        pbase = pix0 + ci * CHUNK

        def grp(g, _):
            s = pl.ds(g * 16, 16)
            wta = w_v[s]
            wtb = w_v[pl.ds(CHUNK + g * 16, 16)]
            wtc = w_v[pl.ds(2 * CHUNK + g * 16, 16)]
            wtd = w_v[pl.ds(3 * CHUNK + g * 16, 16)]
            rloc = jnp.full((16,), g * 16, jnp.int32) + lane
            for c in range(3):
                csp = jnp.full((16,), c, jnp.int32)
                pa = plsc.load_gather(va_v, [rloc, csp])
                pb = plsc.load_gather(vb_v, [rloc, csp])
                pc = plsc.load_gather(vc_v, [rloc, csp])
                pd = plsc.load_gather(vd_v, [rloc, csp])
                outs[c][s] = wta * pa + wtb * pb + wtc * pc + wtd * pd
            return _

        lax.fori_loop(0, GRP, grp, None)
        for c in range(3):
            pltpu.sync_copy(
                outs[c], out_hbm.at[pl.ds((b * C + c) * HW + pbase, CHUNK)])

    idxs_a = tuple(ia_v)
    idxs_b = tuple(ib_v)

    pass1(0, (uc_a, vc_a), idxs_a, wa_v)
    fire(idxs_a)

    def two_chunks(i, _):
        k = 2 * i

        @pl.when(k + 1 < NCHUNK)
        def _a():
            pass1(k + 1, (uc_b, vc_b), idxs_b, wb_v)

        drain(idxs_a)
        pass2(k, wa_v)

        @pl.when(k + 1 < NCHUNK)
        def _b():
            fire(idxs_b)

        @pl.when(k + 2 < NCHUNK)
        def _c():
            pass1(k + 2, (uc_a, vc_a), idxs_a, wa_v)

        @pl.when(k + 1 < NCHUNK)
        def _d():
            drain(idxs_b)
            pass2(k + 1, wb_v)

        @pl.when(k + 2 < NCHUNK)
        def _e():
            fire(idxs_a)

        return _

    lax.fori_loop(0, (NCHUNK + 1) // 2, two_chunks, None)


def kernel(img, depth, T, K, K_inv):
    imgn = img.reshape(B * C * HW)
    uu = jnp.arange(W, dtype=jnp.float32)
    vv0 = jnp.arange(H, dtype=jnp.float32)
    vg, ug = jnp.meshgrid(vv0, uu, indexing="ij")
    ones = jnp.ones_like(ug.reshape(-1))
    grids = jnp.stack([ug.reshape(-1), vg.reshape(-1), ones], axis=0)
    grids = jnp.broadcast_to(grids[None, :, :], (B, 3, HW))
    grids = jnp.matmul(K_inv, grids)
    grids = depth.reshape(B, -1, HW) * grids
    grids = jnp.concatenate(
        [grids, jnp.ones((B, 1, HW), dtype=grids.dtype)], axis=1)
    grids = jnp.matmul(T, grids)
    grids = jnp.matmul(K, grids)
    grids = grids / (grids[:, 2, :][:, None, :] + 1e-16)
    uu_f = grids[:, 0, :].reshape(B * HW)
    vv_f = grids[:, 1, :].reshape(B * HW)

    mesh = plsc.VectorSubcoreMesh(core_axis_name="c", subcore_axis_name="s")
    fn = pl.kernel(
        _sc_warp, mesh=mesh,
        out_type=(jax.ShapeDtypeStruct((B * C * HW,), jnp.float32),
                  jax.ShapeDtypeStruct((B * HW, C), jnp.float32)),
        scratch_types=(
            [pltpu.VMEM((32,), jnp.float32)]
            + [pltpu.VMEM((CHUNK,), jnp.float32)] * 4
            + [[pltpu.VMEM((CHUNK,), jnp.int32)] * 4] * 2
            + [pltpu.VMEM((4 * CHUNK,), jnp.float32)] * 2
            + [pltpu.VMEM((CHUNK, C), jnp.float32)] * 4
            + [pltpu.VMEM((CHUNK,), jnp.float32)] * 3
            + [pltpu.VMEM((CHUNK, C), jnp.float32)]
            + [pltpu.SemaphoreType.DMA]
        ),
        compiler_params=pltpu.CompilerParams(
            needs_layout_passes=False, use_tc_tiling_on_sc=False),
    )
    out, _ = fn(uu_f, vv_f, imgn)
    return out.reshape(B, C, H, W)
